# Initial kernel scaffold; baseline (speedup 1.0000x reference)
import functools

import jax
import jax.numpy as jnp
from jax import lax
from jax.experimental import pallas as pl
from jax.experimental.pallas import tpu as pltpu
from jax.experimental.pallas import tpu_sc as plsc

B, L, D, V = 16384, 128, 32, 31002
PAD_ID, CLS_ID, SEP_ID = 1, 4, 5

NC, NS, LANES = 2, 16, 16
NW = NC * NS
BPW = B // NW
CB = 8
NCH = BPW // CB
TOK = CB * L


def _body(ids_hbm, attn_hbm, table_hbm, out_hbm,
          ids_v, attn_v, idx_v, rows_v, out_v, cnt_s, sem):
    wid = lax.axis_index("s") * NC + lax.axis_index("c")

    @pl.loop(0, NCH)
    def _chunk(c):
        base = wid * BPW + c * CB
        pltpu.sync_copy(ids_hbm.at[pl.ds(base, CB)], ids_v)
        pltpu.sync_copy(attn_hbm.at[pl.ds(base, CB)], attn_v)

        for j in range(CB):
            cnt_vec = jnp.zeros((LANES,), jnp.int32)
            for k in range(L // LANES):
                ids16 = ids_v[j, pl.ds(k * LANES, LANES)]
                attn16 = attn_v[j, pl.ds(k * LANES, LANES)]
                m = (attn16 != 0) & (ids16 != CLS_ID) & (ids16 != SEP_ID)
                idx_v[j, pl.ds(k * LANES, LANES)] = jnp.where(m, ids16, PAD_ID)
                cnt_vec = cnt_vec + m.astype(jnp.int32)
            cnt_s[j] = jnp.maximum(jnp.sum(cnt_vec), 1)

        copies = [
            pltpu.async_copy(
                table_hbm.at[idx_v.at[j]],
                rows_v.at[pl.ds(j * L, L)],
                sem,
            )
            for j in range(CB)
        ]
        for cp in copies:
            cp.wait()

        for j in range(CB):
            zero = jnp.zeros((LANES,), jnp.float32)

            def _step(t, carry, j=j):
                a0, a1 = carry
                for u in range(16):
                    tok = j * L + t * 16 + u
                    a0 = a0 + rows_v[tok, pl.ds(0, LANES)]
                    a1 = a1 + rows_v[tok, pl.ds(LANES, LANES)]
                return a0, a1

            a0, a1 = lax.fori_loop(0, L // 16, _step, (zero, zero))
            cf = cnt_s[j].astype(jnp.float32)
            out_v[j, pl.ds(0, LANES)] = a0 / cf
            out_v[j, pl.ds(LANES, LANES)] = a1 / cf

        pltpu.sync_copy(out_v, out_hbm.at[pl.ds(base, CB)])


@jax.jit
def _run(ids, attn, table):
    mesh = plsc.VectorSubcoreMesh(core_axis_name="c", subcore_axis_name="s")
    f = pl.kernel(
        _body,
        out_type=jax.ShapeDtypeStruct((B, D), jnp.float32),
        mesh=mesh,
        scratch_types=[
            pltpu.VMEM((CB, L), jnp.int32),
            pltpu.VMEM((CB, L), jnp.int32),
            pltpu.VMEM((CB, L), jnp.int32),
            pltpu.VMEM((TOK, D), jnp.float32),
            pltpu.VMEM((CB, D), jnp.float32),
            pltpu.SMEM((CB,), jnp.int32),
            pltpu.SemaphoreType.DMA,
        ],
    )
    return f(ids, attn, table)


def kernel(input_ids, attention_mask, table):
    ids = input_ids.astype(jnp.int32)
    attn = attention_mask.astype(jnp.int32)
    return _run(ids, attn, table)


# speedup vs baseline: 24.8716x; 24.8716x over previous
import functools

import jax
import jax.numpy as jnp
from jax import lax
from jax.experimental import pallas as pl
from jax.experimental.pallas import tpu as pltpu
from jax.experimental.pallas import tpu_sc as plsc

B, L, D, V = 16384, 128, 32, 31002
PAD_ID, CLS_ID, SEP_ID = 1, 4, 5

NC, NS, LANES = 2, 16, 16
NW = NC * NS
BPW = B // NW
CB = 8
NCH = BPW // CB
TOK = CB * L


def _body(ids_hbm, attn_hbm, table_hbm, out_hbm,
          ids_v, attn_v, idx_v, rows_v, out_v, cnt_s, sem):
    wid = lax.axis_index("s") * NC + lax.axis_index("c")

    @pl.loop(0, NCH)
    def _chunk(c):
        base = wid * BPW + c * CB
        pltpu.sync_copy(ids_hbm.at[pl.ds(base, CB)], ids_v)
        pltpu.sync_copy(attn_hbm.at[pl.ds(base, CB)], attn_v)

        for j in range(CB):
            cnt_vec = jnp.zeros((LANES,), jnp.int32)
            for k in range(L // LANES):
                ids16 = ids_v[j, pl.ds(k * LANES, LANES)]
                attn16 = attn_v[j, pl.ds(k * LANES, LANES)]
                m = (attn16 != 0) & (ids16 != CLS_ID) & (ids16 != SEP_ID)
                idx_v[j, pl.ds(k * LANES, LANES)] = jnp.where(m, ids16, PAD_ID)
                cnt_vec = cnt_vec + m.astype(jnp.int32)
            cnt_s[j] = jnp.maximum(jnp.sum(cnt_vec), 1)

        copies = [
            pltpu.async_copy(
                table_hbm.at[idx_v.at[j]],
                rows_v.at[pl.ds(j * L, L)],
                sem,
            )
            for j in range(CB)
        ]
        for cp in copies:
            cp.wait()

        for j in range(CB):
            zero = jnp.zeros((LANES,), jnp.float32)

            def _step(t, carry, j=j):
                a0, a1 = carry
                for u in range(16):
                    tok = j * L + t * 16 + u
                    a0 = a0 + rows_v[tok, pl.ds(0, LANES)]
                    a1 = a1 + rows_v[tok, pl.ds(LANES, LANES)]
                return a0, a1

            a0, a1 = lax.fori_loop(0, L // 16, _step, (zero, zero))
            cf = cnt_s[j].astype(jnp.float32)
            out_v[j, pl.ds(0, LANES)] = a0 / cf
            out_v[j, pl.ds(LANES, LANES)] = a1 / cf

        pltpu.sync_copy(out_v, out_hbm.at[pl.ds(base, CB)])


@jax.jit
def _run(ids, attn, table):
    mesh = plsc.VectorSubcoreMesh(core_axis_name="c", subcore_axis_name="s")
    f = pl.kernel(
        _body,
        out_type=jax.ShapeDtypeStruct((B, D), jnp.float32),
        mesh=mesh,
        scratch_types=[
            pltpu.VMEM((CB, L), jnp.int32),
            pltpu.VMEM((CB, L), jnp.int32),
            pltpu.VMEM((CB, L), jnp.int32),
            pltpu.VMEM((TOK, D), jnp.float32),
            pltpu.VMEM((CB, D), jnp.float32),
            pltpu.SMEM((CB,), jnp.int32),
            pltpu.SemaphoreType.DMA,
        ],
        compiler_params=pltpu.CompilerParams(
            use_tc_tiling_on_sc=False, needs_layout_passes=False
        ),
    )
    return f(ids, attn, table)


def kernel(input_ids, attention_mask, table):
    ids = input_ids.astype(jnp.int32)
    attn = attention_mask.astype(jnp.int32)
    return _run(ids, attn, table)


# speedup vs baseline: 32.9329x; 1.3241x over previous
import jax
import jax.numpy as jnp
from jax import lax
from jax.experimental import pallas as pl
from jax.experimental.pallas import tpu as pltpu
from jax.experimental.pallas import tpu_sc as plsc

B, L, D, V = 16384, 128, 32, 31002
PAD_ID, CLS_ID, SEP_ID = 1, 4, 5

NC, LANES = 2, 16
NW = 32
BPW = B // NW
CB = 8
NCH = BPW // CB
TOK = CB * L


def _load_remap_fire(ids_hbm, attn_hbm, table_hbm, ids_v, attn_v, idx_v,
                     rows_v, cnt_s, sem, base, p):
    pltpu.sync_copy(ids_hbm.at[pl.ds(base, CB)], ids_v.at[p])
    pltpu.sync_copy(attn_hbm.at[pl.ds(base, CB)], attn_v.at[p])
    for j in range(CB):
        cnt_vec = jnp.zeros((LANES,), jnp.int32)
        for k in range(L // LANES):
            ids16 = ids_v[p, j, pl.ds(k * LANES, LANES)]
            attn16 = attn_v[p, j, pl.ds(k * LANES, LANES)]
            m = (attn16 != 0) & (ids16 != CLS_ID) & (ids16 != SEP_ID)
            idx_v[p, j, pl.ds(k * LANES, LANES)] = jnp.where(m, ids16, PAD_ID)
            cnt_vec = cnt_vec + m.astype(jnp.int32)
        cnt_s[p * CB + j] = jnp.maximum(jnp.sum(cnt_vec), 1)
    for j in range(CB):
        pltpu.async_copy(
            table_hbm.at[idx_v.at[p].at[j]],
            rows_v.at[p].at[pl.ds(j * L, L)],
            sem,
        )


def _drain_pool_store(table_hbm, out_hbm, idx_v, rows_v, out_v, cnt_s,
                      sem, sem_out, base, p, first):
    for j in range(CB):
        pltpu.make_async_copy(
            table_hbm.at[idx_v.at[p].at[j]],
            rows_v.at[p].at[pl.ds(j * L, L)],
            sem,
        ).wait()
    @pl.when(jnp.logical_not(first))
    def _drain_out():
        pltpu.make_async_copy(
            out_v.at[p], out_hbm.at[pl.ds(base - 2 * CB, CB)], sem_out
        ).wait()
    for j in range(CB):
        zero = jnp.zeros((LANES,), jnp.float32)

        def _step(t, carry, j=j):
            a0, a1 = carry
            for u in range(16):
                tok = j * L + t * 16 + u
                a0 = a0 + rows_v[p, tok, pl.ds(0, LANES)]
                a1 = a1 + rows_v[p, tok, pl.ds(LANES, LANES)]
            return a0, a1

        a0, a1 = lax.fori_loop(0, L // 16, _step, (zero, zero))
        cf = cnt_s[p * CB + j].astype(jnp.float32)
        out_v[p, j, pl.ds(0, LANES)] = a0 / cf
        out_v[p, j, pl.ds(LANES, LANES)] = a1 / cf
    pltpu.async_copy(out_v.at[p], out_hbm.at[pl.ds(base, CB)], sem_out)


def _body(ids_hbm, attn_hbm, table_hbm, out_hbm,
          ids_v, attn_v, idx_v, rows_v, out_v, cnt_s, sem0, sem1,
          semo0, semo1):
    wid = lax.axis_index("s") * NC + lax.axis_index("c")
    wbase = wid * BPW
    sems = (sem0, sem1)
    sems_out = (semo0, semo1)

    _load_remap_fire(ids_hbm, attn_hbm, table_hbm, ids_v, attn_v,
                     idx_v, rows_v, cnt_s, sems[0], wbase, 0)
    _load_remap_fire(ids_hbm, attn_hbm, table_hbm, ids_v, attn_v,
                     idx_v, rows_v, cnt_s, sems[1], wbase + CB, 1)

    @pl.loop(0, NCH // 2)
    def _iter(i):
        c0 = i * 2
        for p in range(2):
            c = c0 + p
            base = wbase + c * CB
            _drain_pool_store(table_hbm, out_hbm, idx_v, rows_v, out_v,
                              cnt_s, sems[p], sems_out[p], base, p, i == 0)

            @pl.when(c + 2 < NCH)
            def _fire(c=c, p=p):
                _load_remap_fire(ids_hbm, attn_hbm, table_hbm, ids_v, attn_v,
                                 idx_v, rows_v, cnt_s, sems[p],
                                 wbase + (c + 2) * CB, p)

    for p in range(2):
        pltpu.make_async_copy(
            out_v.at[p],
            out_hbm.at[pl.ds(wbase + (NCH - 2 + p) * CB, CB)],
            sems_out[p],
        ).wait()


@jax.jit
def _run(ids, attn, table):
    mesh = plsc.VectorSubcoreMesh(core_axis_name="c", subcore_axis_name="s")
    f = pl.kernel(
        _body,
        out_type=jax.ShapeDtypeStruct((B, D), jnp.float32),
        mesh=mesh,
        scratch_types=[
            pltpu.VMEM((2, CB, L), jnp.int32),
            pltpu.VMEM((2, CB, L), jnp.int32),
            pltpu.VMEM((2, CB, L), jnp.int32),
            pltpu.VMEM((2, TOK, D), jnp.float32),
            pltpu.VMEM((2, CB, D), jnp.float32),
            pltpu.SMEM((2 * CB,), jnp.int32),
            pltpu.SemaphoreType.DMA,
            pltpu.SemaphoreType.DMA,
            pltpu.SemaphoreType.DMA,
            pltpu.SemaphoreType.DMA,
        ],
        compiler_params=pltpu.CompilerParams(
            use_tc_tiling_on_sc=False, needs_layout_passes=False
        ),
    )
    return f(ids, attn, table)


def kernel(input_ids, attention_mask, table):
    ids = input_ids.astype(jnp.int32)
    attn = attention_mask.astype(jnp.int32)
    return _run(ids, attn, table)
